# baseline (speedup 1.0000x reference)
import jax
import jax.numpy as jnp
from jax.experimental import pallas as pl
from jax.experimental.pallas import tpu as pltpu

S_BLK = 2048


def _add_kernel(x_ref, t_ref, o_ref):
    o_ref[0] = x_ref[0] + t_ref[...]


def kernel(inputs, pos_table):
    batch, seq, dim = inputs.shape
    grid = (seq // S_BLK, batch)
    return pl.pallas_call(
        _add_kernel,
        grid=grid,
        in_specs=[
            pl.BlockSpec((1, S_BLK, dim), lambda i, b: (b, i, 0)),
            pl.BlockSpec((S_BLK, dim), lambda i, b: (i, 0)),
        ],
        out_specs=pl.BlockSpec((1, S_BLK, dim), lambda i, b: (b, i, 0)),
        out_shape=jax.ShapeDtypeStruct(inputs.shape, inputs.dtype),
        compiler_params=pltpu.CompilerParams(
            dimension_semantics=("parallel", "parallel"),
        ),
    )(inputs, pos_table)
